# baseline (speedup 1.0000x reference)
import functools
import math

import jax
import jax.numpy as jnp
from jax import lax
from jax.experimental import pallas as pl
from jax.experimental.pallas import tpu as pltpu
from jax.experimental.pallas import tpu_sc as plsc

D_MODEL = 64
SCALE = math.sqrt(D_MODEL)
NC = 2
NS = 16
NW = NC * NS
L = 16


@functools.partial(jax.jit, static_argnums=())
def _embed_flat(xf, table):
    B = xf.shape[0]
    assert B % (8 * NW) == 0
    b_per_w = B // NW
    CH = 128
    n_chunks = b_per_w // CH

    mesh = plsc.VectorSubcoreMesh(core_axis_name="c", subcore_axis_name="s")

    @functools.partial(
        pl.kernel,
        mesh=mesh,
        out_type=jax.ShapeDtypeStruct((B, D_MODEL), jnp.float32),
        compiler_params=pltpu.CompilerParams(use_tc_tiling_on_sc=False),
        scratch_types=[
            pltpu.VMEM((b_per_w,), jnp.int32),
            pltpu.VMEM((CH, D_MODEL), jnp.float32),
            pltpu.SemaphoreType.DMA,
        ],
    )
    def k(x_hbm, table_hbm, out_hbm, idx_v, rows_v, sem):
        wid = lax.axis_index("s") * NC + lax.axis_index("c")
        base = wid * b_per_w
        pltpu.sync_copy(x_hbm.at[pl.ds(base, b_per_w)], idx_v)

        def chunk_body(g, carry):
            off = g * CH
            pltpu.async_copy(
                table_hbm.at[idx_v.at[pl.ds(off, CH)]], rows_v, sem
            ).wait()

            def scale_row(r, c2):
                for c in range(D_MODEL // L):
                    rows_v[r, pl.ds(c * L, L)] = rows_v[r, pl.ds(c * L, L)] * SCALE
                return c2

            lax.fori_loop(0, CH, scale_row, 0, unroll=2)
            pltpu.sync_copy(rows_v, out_hbm.at[pl.ds(base + off, CH)])
            return carry

        lax.fori_loop(0, n_chunks, chunk_body, 0)

    return k(xf, table)


def kernel(x, table):
    n, s = x.shape
    out = _embed_flat(x.reshape(n * s), table)
    return out.reshape(n, s, D_MODEL)


# speedup vs baseline: 1.1461x; 1.1461x over previous
import functools
import math

import jax
import jax.numpy as jnp
from jax import lax
from jax.experimental import pallas as pl
from jax.experimental.pallas import tpu as pltpu
from jax.experimental.pallas import tpu_sc as plsc

D_MODEL = 64
SCALE = math.sqrt(D_MODEL)
NC = 2
NS = 16
NW = NC * NS
L = 16
CH = 128
NB = 4


def _embed_flat(xf, table):
    B = xf.shape[0]
    assert B % (NW * CH) == 0
    b_per_w = B // NW
    n_chunks = b_per_w // CH

    mesh = plsc.VectorSubcoreMesh(core_axis_name="c", subcore_axis_name="s")

    @functools.partial(
        pl.kernel,
        mesh=mesh,
        out_type=jax.ShapeDtypeStruct((B, D_MODEL), jnp.float32),
        compiler_params=pltpu.CompilerParams(use_tc_tiling_on_sc=False),
        scratch_types=[
            pltpu.VMEM((b_per_w,), jnp.int32),
            pltpu.VMEM((NB, CH, D_MODEL), jnp.float32),
            [pltpu.SemaphoreType.DMA] * NB,
            [pltpu.SemaphoreType.DMA] * NB,
        ],
    )
    def k(x_hbm, table_hbm, out_hbm, idx_v, bufs, gsems, ssems):
        wid = lax.axis_index("s") * NC + lax.axis_index("c")
        base = wid * b_per_w
        pltpu.sync_copy(x_hbm.at[pl.ds(base, b_per_w)], idx_v)

        def gather_start(g, slot):
            pltpu.async_copy(
                table_hbm.at[idx_v.at[pl.ds(g * CH, CH)]],
                bufs.at[slot],
                gsems[slot],
            )

        def gather_wait(slot):
            pltpu.make_async_copy(
                table_hbm.at[idx_v.at[pl.ds(0, CH)]],
                bufs.at[slot],
                gsems[slot],
            ).wait()

        def store_start(g, slot):
            pltpu.async_copy(
                bufs.at[slot],
                out_hbm.at[pl.ds(base + g * CH, CH)],
                ssems[slot],
            )

        def store_wait(slot):
            pltpu.make_async_copy(
                bufs.at[slot],
                out_hbm.at[pl.ds(base, CH)],
                ssems[slot],
            ).wait()

        gather_start(0, 0)
        gather_start(1, 1)

        @pl.loop(0, n_chunks, step=NB)
        def superstep(s):
            for b in range(NB):
                g = s + b
                gather_wait(b)

                @plsc.parallel_loop(0, CH, unroll=4)
                def scale_row(r):
                    for c in range(D_MODEL // L):
                        bufs[b, r, pl.ds(c * L, L)] = (
                            bufs[b, r, pl.ds(c * L, L)] * SCALE
                        )

                store_start(g, b)
                nslot = (b + 2) % NB

                @pl.when(g + 2 < n_chunks)
                def _():
                    @pl.when(g >= 2)
                    def _():
                        store_wait(nslot)

                    gather_start(g + 2, nslot)

        for b in range(NB):
            store_wait(b)

    return k(xf, table)


def kernel(x, table):
    n, s = x.shape
    out = _embed_flat(x.reshape(n * s), table)
    return out.reshape(n, s, D_MODEL)
